# baseline (speedup 1.0000x reference)
import functools

import jax
import jax.numpy as jnp
from jax import lax
from jax.experimental import pallas as pl
from jax.experimental.pallas import tpu as pltpu
from jax.experimental.pallas import tpu_sc as plsc

H = 128
N = 50000
E = 500000

NC = 2
NS = 16
LANES = 16

K_CHUNKS = 4
C = 12544
NP = K_CHUNKS * C

EP = 524288
S_TILE = EP // NS
EB = 1024
NB = S_TILE // EB
GB = 64
SH = GB.bit_length() - 1
NROWS = (EB + GB) // GB
TROW = NROWS
WB = 56
SPT = C // NS


def _mm_body(k, x_ref, p_ref, b_ref, *outs):
    acc = jnp.dot(x_ref[...], p_ref[...], preferred_element_type=jnp.float32)
    outs[0][...] = acc[:, :H] + b_ref[...]
    for j in range(1, k):
        outs[j][...] = acc[:, j * H:(j + 1) * H]


def _make_mm(k):
    mblk = 1024
    return pl.pallas_call(
        functools.partial(_mm_body, k),
        grid=(NP // mblk,),
        in_specs=[
            pl.BlockSpec((mblk, H), lambda i: (i, 0)),
            pl.BlockSpec((H, k * H), lambda i: (0, 0)),
            pl.BlockSpec((1, H), lambda i: (0, 0)),
        ],
        out_specs=[pl.BlockSpec((mblk, H), lambda i: (i, 0))] * k,
        out_shape=[jax.ShapeDtypeStruct((NP, H), jnp.float32)] * k,
    )


def _make_sc():
    mesh = plsc.VectorSubcoreMesh(
        core_axis_name="c", subcore_axis_name="s",
        num_cores=NC, num_subcores=NS)
    scratch = [
        pltpu.VMEM_SHARED((C, H), jnp.float32),
        pltpu.VMEM((3 * EB,), jnp.int32),
        pltpu.VMEM((NROWS + 1, GB), jnp.int32),
        pltpu.VMEM((NROWS + 1, GB), jnp.int32),
        pltpu.VMEM((NROWS + 1, GB + LANES), jnp.float32),
        pltpu.VMEM((GB, H), jnp.float32),
        pltpu.VMEM((GB, H), jnp.float32),
        pltpu.SemaphoreType.DMA,
        pltpu.SemaphoreType.DMA,
        pltpu.SemaphoreType.DMA,
        pltpu.SemaphoreType.DMA,
    ]

    def body(*refs):
        denses = refs[0:4]
        relrefs = refs[4:16]
        outs = refs[16:20]
        (acc, ebuf, csrc, cdst, cw, rows0, rows1,
         g0sem, g1sem, s0sem, s1sem) = refs[20:]

        cid = lax.axis_index("c")
        sid = lax.axis_index("s")
        zf16 = jnp.zeros((LANES,), jnp.float32)

        def scale(b, rbuf):
            def srow(g2, _):
                for u in range(2):
                    g = g2 * 2 + u
                    wv16 = cw[b, pl.ds(g, LANES)]
                    wsp = jnp.full((LANES,), wv16[0], jnp.float32)
                    for h in range(H // LANES):
                        sl = pl.ds(h * LANES, LANES)
                        rbuf[g, sl] = rbuf[g, sl] * wsp
                return 0
            lax.fori_loop(0, GB // 2, srow, 0)

        def fire_all(nfire, m_hbm):
            @pl.when(nfire > 0)
            def _():
                pltpu.async_copy(m_hbm.at[csrc.at[0]], rows0, g0sem)

            def pair(k, _):
                b0 = 2 * k
                pltpu.make_async_copy(m_hbm.at[csrc.at[b0]], rows0,
                                      g0sem).wait()

                @pl.when(b0 + 1 < nfire)
                def _():
                    @pl.when(b0 >= 1)
                    def _():
                        pltpu.make_async_copy(
                            rows1, acc.at[cdst.at[0]], s1sem).wait()
                    pltpu.async_copy(m_hbm.at[csrc.at[b0 + 1]], rows1,
                                     g1sem)
                scale(b0, rows0)
                pltpu.async_copy(rows0, acc.at[cdst.at[b0]], s0sem,
                                 add=True)

                @pl.when(b0 + 1 < nfire)
                def _():
                    pltpu.make_async_copy(m_hbm.at[csrc.at[b0 + 1]],
                                          rows1, g1sem).wait()

                    @pl.when(b0 + 2 < nfire)
                    def _():
                        pltpu.make_async_copy(
                            rows0, acc.at[cdst.at[0]], s0sem).wait()
                        pltpu.async_copy(m_hbm.at[csrc.at[b0 + 2]],
                                         rows0, g0sem)
                    scale(b0 + 1, rows1)
                    pltpu.async_copy(rows1, acc.at[cdst.at[b0 + 1]],
                                     s1sem, add=True)
                return 0
            lax.fori_loop(0, (nfire + 1) // 2, pair, 0)

            @pl.when(nfire >= 1)
            def _():
                pltpu.make_async_copy(rows0, acc.at[cdst.at[0]],
                                      s0sem).wait()

            @pl.when(nfire >= 2)
            def _():
                pltpu.make_async_copy(rows1, acc.at[cdst.at[0]],
                                      s1sem).wait()

        lane = lax.iota(jnp.int32, LANES)
        ones16 = jnp.ones((LANES,), jnp.int32)
        zeros16 = jnp.zeros((LANES,), jnp.int32)
        trow_v = jnp.full((LANES,), TROW, jnp.int32)
        c127 = jnp.full((LANES,), GB - 1, jnp.int32)

        def scan_rel(m_hbm, ep_hbm, lo):
            lov = jnp.full((LANES,), lo, jnp.int32)
            hiv = lov + jnp.full((LANES,), C, jnp.int32)

            def bbody(b, cnt):
                base = (sid * NB + b) * (3 * EB)
                pltpu.sync_copy(ep_hbm.at[pl.ds(base, 3 * EB)], ebuf)

                def step(i, cnt):
                    off = i * LANES
                    sv = ebuf[pl.ds(off, LANES)]
                    dv = ebuf[pl.ds(EB + off, LANES)]
                    wv = plsc.bitcast(ebuf[pl.ds(2 * EB + off, LANES)],
                                      jnp.float32)
                    msk = (dv >= lov) & (dv < hiv)
                    pc = plsc.cumsum(msk.astype(jnp.int32))
                    cntv = jnp.full((LANES,), cnt, jnp.int32)
                    tgt = jnp.where(msk, cntv + pc - ones16, zeros16)
                    thi = jnp.where(msk, lax.shift_right_logical(tgt, SH),
                                    trow_v)
                    tlo = jnp.where(msk, tgt & c127, lane)
                    plsc.store_scatter(csrc, [thi, tlo], sv)
                    plsc.store_scatter(cdst, [thi, tlo], dv - lov)
                    plsc.store_scatter(cw, [thi, tlo], wv)
                    pcv = plsc.all_reduce_population_count(msk)
                    return cnt + pcv[0]
                cnt = lax.fori_loop(0, EB // LANES, step, cnt)

                nfire = cnt // GB
                fire_all(nfire, m_hbm)

                @pl.when(nfire > 0)
                def _move():
                    for h in range(GB // LANES):
                        sl = pl.ds(h * LANES, LANES)
                        csrc[0, sl] = csrc[nfire, sl]
                        cdst[0, sl] = cdst[nfire, sl]
                        cw[0, sl] = cw[nfire, sl]
                return cnt - nfire * GB

            cnt = lax.fori_loop(0, NB, bbody, jnp.int32(0))

            cntv = jnp.full((LANES,), cnt, jnp.int32)
            for t in range(GB // LANES):
                colv = jnp.full((LANES,), t * LANES, jnp.int32) + lane
                sel = colv >= cntv
                zhi = jnp.where(sel, zeros16, trow_v)
                zlo = jnp.where(sel, colv, lane)
                plsc.store_scatter(csrc, [zhi, zlo], zeros16)
                plsc.store_scatter(cdst, [zhi, zlo], zeros16)
                plsc.store_scatter(cw, [zhi, zlo], zf16)

            @pl.when(cnt > 0)
            def _tail():
                pltpu.async_copy(m_hbm.at[csrc.at[0]], rows0,
                                 g0sem).wait()
                scale(0, rows0)
                pltpu.sync_copy(rows0, acc.at[cdst.at[0]], add=True)

        groups = [
            (denses[0], outs[0],
             [(relrefs[0], relrefs[1]), (relrefs[2], relrefs[3]),
              (relrefs[4], relrefs[5])]),
            (denses[1], outs[1], [(relrefs[6], relrefs[7])]),
            (denses[2], outs[2], [(relrefs[8], relrefs[9])]),
            (denses[3], outs[3], [(relrefs[10], relrefs[11])]),
        ]
        for dense, out, rels in groups:
            for j in range(K_CHUNKS // NC):
                lo = (cid + NC * j) * C
                abase = sid * SPT

                pltpu.sync_copy(dense.at[pl.ds(lo + abase, SPT)],
                                acc.at[pl.ds(abase, SPT)])
                plsc.subcore_barrier()

                for (m_hbm, ep_hbm) in rels:
                    scan_rel(m_hbm, ep_hbm, lo)
                plsc.subcore_barrier()

                pltpu.sync_copy(acc.at[pl.ds(abase, SPT)],
                                out.at[pl.ds(lo + abase, SPT)])
                plsc.subcore_barrier()

    return pl.kernel(
        body,
        out_type=[jax.ShapeDtypeStruct((NP, H), jnp.float32)] * 4,
        mesh=mesh,
        scratch_types=scratch,
        compiler_params=pltpu.CompilerParams(needs_layout_passes=False),
    )


_mm4 = _make_mm(4)
_mm2 = _make_mm(2)
_sc_all = _make_sc()


def _pad_x(x):
    return jnp.pad(x, ((0, NP - N), (0, 0)))


def _pack_edges(ei, ew):
    pad = EP - E
    s = jnp.concatenate([ei[0], jnp.zeros((pad,), jnp.int32)])
    d = jnp.concatenate([ei[1], jnp.full((pad,), jnp.int32(1 << 30))])
    w = lax.bitcast_convert_type(
        jnp.concatenate([ew, jnp.zeros((pad,), jnp.float32)]), jnp.int32)
    blk = jnp.stack(
        [s.reshape(-1, EB), d.reshape(-1, EB), w.reshape(-1, EB)], axis=1)
    return blk.reshape(-1)


def kernel(x_app, x_sys, x_bnd, x_cmp, ei_app_sys, ew_app_sys, ei_sys_app,
           ew_sys_app, ei_app_bnd, ew_app_bnd, ei_bnd_app, ew_bnd_app,
           ei_app_cmp, ew_app_cmp, ei_cmp_app, ew_cmp_app, W_self_app,
           b_self_app, W_self_sys, b_self_sys, W_self_bnd, b_self_bnd,
           W_self_cmp, b_self_cmp, Wn_app_sys, Ws_app_sys, b_app_sys,
           Wn_sys_app, Ws_sys_app, b_sys_app, Wn_app_bnd, Ws_app_bnd,
           b_app_bnd, Wn_bnd_app, Ws_bnd_app, b_bnd_app, Wn_app_cmp,
           Ws_app_cmp, b_app_cmp, Wn_cmp_app, Ws_cmp_app, b_cmp_app):
    P_app = jnp.concatenate(
        [(W_self_app + Ws_sys_app + Ws_bnd_app + Ws_cmp_app).T,
         Wn_app_sys.T, Wn_app_bnd.T, Wn_app_cmp.T], axis=1)
    be_app = (b_self_app + b_sys_app + b_bnd_app + b_cmp_app).reshape(1, H)
    P_sys = jnp.concatenate([(W_self_sys + Ws_app_sys).T, Wn_sys_app.T],
                            axis=1)
    be_sys = (b_self_sys + b_app_sys).reshape(1, H)
    P_bnd = jnp.concatenate([(W_self_bnd + Ws_app_bnd).T, Wn_bnd_app.T],
                            axis=1)
    be_bnd = (b_self_bnd + b_app_bnd).reshape(1, H)
    P_cmp = jnp.concatenate([(W_self_cmp + Ws_app_cmp).T, Wn_cmp_app.T],
                            axis=1)
    be_cmp = (b_self_cmp + b_app_cmp).reshape(1, H)

    dense_app, m_app_sys, m_app_bnd, m_app_cmp = _mm4(
        _pad_x(x_app), P_app, be_app)
    dense_sys, m_sys_app = _mm2(_pad_x(x_sys), P_sys, be_sys)
    dense_bnd, m_bnd_app = _mm2(_pad_x(x_bnd), P_bnd, be_bnd)
    dense_cmp, m_cmp_app = _mm2(_pad_x(x_cmp), P_cmp, be_cmp)

    ep_sa = _pack_edges(ei_sys_app, ew_sys_app)
    ep_ba = _pack_edges(ei_bnd_app, ew_bnd_app)
    ep_ca = _pack_edges(ei_cmp_app, ew_cmp_app)
    ep_as = _pack_edges(ei_app_sys, ew_app_sys)
    ep_ab = _pack_edges(ei_app_bnd, ew_app_bnd)
    ep_ac = _pack_edges(ei_app_cmp, ew_app_cmp)

    out_app, out_sys, out_bnd, out_cmp = _sc_all(
        dense_app, dense_sys, dense_bnd, dense_cmp,
        m_sys_app, ep_sa, m_bnd_app, ep_ba, m_cmp_app, ep_ca,
        m_app_sys, ep_as, m_app_bnd, ep_ab, m_app_cmp, ep_ac)

    return (out_app[:N], out_sys[:N], out_bnd[:N], out_cmp[:N])


# speedup vs baseline: 1.0280x; 1.0280x over previous
import functools

import jax
import jax.numpy as jnp
from jax import lax
from jax.experimental import pallas as pl
from jax.experimental.pallas import tpu as pltpu
from jax.experimental.pallas import tpu_sc as plsc

H = 128
N = 50000
E = 500000

NC = 2
NS = 16
LANES = 16

K_CHUNKS = 4
C = 12544
NP = K_CHUNKS * C

EP = 524288
S_TILE = EP // NS
EB = 1024
NB = S_TILE // EB
GB = 64
SH = GB.bit_length() - 1
NROWS = (EB + GB) // GB
TROW = NROWS
WB = 56
SPT = C // NS


def _mm_body(k, x_ref, p_ref, b_ref, *outs):
    acc = jnp.dot(x_ref[...], p_ref[...], preferred_element_type=jnp.float32)
    outs[0][...] = acc[:, :H] + b_ref[...]
    for j in range(1, k):
        outs[j][...] = acc[:, j * H:(j + 1) * H]


def _make_mm(k):
    mblk = 1024
    return pl.pallas_call(
        functools.partial(_mm_body, k),
        grid=(NP // mblk,),
        in_specs=[
            pl.BlockSpec((mblk, H), lambda i: (i, 0)),
            pl.BlockSpec((H, k * H), lambda i: (0, 0)),
            pl.BlockSpec((1, H), lambda i: (0, 0)),
        ],
        out_specs=[pl.BlockSpec((mblk, H), lambda i: (i, 0))] * k,
        out_shape=[jax.ShapeDtypeStruct((NP, H), jnp.float32)] * k,
    )


def _make_sc(n_rels):
    mesh = plsc.VectorSubcoreMesh(
        core_axis_name="c", subcore_axis_name="s",
        num_cores=NC, num_subcores=NS)
    scratch = [
        pltpu.VMEM_SHARED((C, H), jnp.float32),
        pltpu.VMEM((3 * EB,), jnp.int32),
        pltpu.VMEM((NROWS + 1, GB), jnp.int32),
        pltpu.VMEM((NROWS + 1, GB), jnp.int32),
        pltpu.VMEM((NROWS + 1, GB + LANES), jnp.float32),
        pltpu.VMEM((GB, H), jnp.float32),
        pltpu.VMEM((GB, H), jnp.float32),
        pltpu.SemaphoreType.DMA,
        pltpu.SemaphoreType.DMA,
        pltpu.SemaphoreType.DMA,
        pltpu.SemaphoreType.DMA,
    ]

    def body(*refs):
        dense = refs[0]
        rels = [refs[1 + 2 * j:3 + 2 * j] for j in range(n_rels)]
        out = refs[1 + 2 * n_rels]
        (acc, ebuf, csrc, cdst, cw, rows0, rows1,
         g0sem, g1sem, s0sem, s1sem) = refs[2 + 2 * n_rels:]

        cid = lax.axis_index("c")
        sid = lax.axis_index("s")
        zf16 = jnp.zeros((LANES,), jnp.float32)

        def scale(b, rbuf):
            def srow(g2, _):
                for u in range(2):
                    g = g2 * 2 + u
                    wv16 = cw[b, pl.ds(g, LANES)]
                    wsp = jnp.full((LANES,), wv16[0], jnp.float32)
                    for h in range(H // LANES):
                        sl = pl.ds(h * LANES, LANES)
                        rbuf[g, sl] = rbuf[g, sl] * wsp
                return 0
            lax.fori_loop(0, GB // 2, srow, 0)

        def fire_all(nfire, m_hbm):
            @pl.when(nfire > 0)
            def _():
                pltpu.async_copy(m_hbm.at[csrc.at[0]], rows0, g0sem)

            def pair(k, _):
                b0 = 2 * k
                pltpu.make_async_copy(m_hbm.at[csrc.at[b0]], rows0,
                                      g0sem).wait()

                @pl.when(b0 + 1 < nfire)
                def _():
                    @pl.when(b0 >= 1)
                    def _():
                        pltpu.make_async_copy(
                            rows1, acc.at[cdst.at[0]], s1sem).wait()
                    pltpu.async_copy(m_hbm.at[csrc.at[b0 + 1]], rows1,
                                     g1sem)
                scale(b0, rows0)
                pltpu.async_copy(rows0, acc.at[cdst.at[b0]], s0sem,
                                 add=True)

                @pl.when(b0 + 1 < nfire)
                def _():
                    pltpu.make_async_copy(m_hbm.at[csrc.at[b0 + 1]],
                                          rows1, g1sem).wait()

                    @pl.when(b0 + 2 < nfire)
                    def _():
                        pltpu.make_async_copy(
                            rows0, acc.at[cdst.at[0]], s0sem).wait()
                        pltpu.async_copy(m_hbm.at[csrc.at[b0 + 2]],
                                         rows0, g0sem)
                    scale(b0 + 1, rows1)
                    pltpu.async_copy(rows1, acc.at[cdst.at[b0 + 1]],
                                     s1sem, add=True)
                return 0
            lax.fori_loop(0, (nfire + 1) // 2, pair, 0)

            @pl.when(nfire >= 1)
            def _():
                pltpu.make_async_copy(rows0, acc.at[cdst.at[0]],
                                      s0sem).wait()

            @pl.when(nfire >= 2)
            def _():
                pltpu.make_async_copy(rows1, acc.at[cdst.at[0]],
                                      s1sem).wait()

        lane = lax.iota(jnp.int32, LANES)
        ones16 = jnp.ones((LANES,), jnp.int32)
        zeros16 = jnp.zeros((LANES,), jnp.int32)
        trow_v = jnp.full((LANES,), TROW, jnp.int32)
        c127 = jnp.full((LANES,), GB - 1, jnp.int32)

        def scan_rel(m_hbm, ep_hbm, lo):
            lov = jnp.full((LANES,), lo, jnp.int32)
            hiv = lov + jnp.full((LANES,), C, jnp.int32)

            def bbody(b, cnt):
                base = (sid * NB + b) * (3 * EB)
                pltpu.sync_copy(ep_hbm.at[pl.ds(base, 3 * EB)], ebuf)

                def step(i, cnt):
                    off = i * LANES
                    sv = ebuf[pl.ds(off, LANES)]
                    dv = ebuf[pl.ds(EB + off, LANES)]
                    wv = plsc.bitcast(ebuf[pl.ds(2 * EB + off, LANES)],
                                      jnp.float32)
                    msk = (dv >= lov) & (dv < hiv)
                    pc = plsc.cumsum(msk.astype(jnp.int32))
                    cntv = jnp.full((LANES,), cnt, jnp.int32)
                    tgt = jnp.where(msk, cntv + pc - ones16, zeros16)
                    thi = jnp.where(msk, lax.shift_right_logical(tgt, SH),
                                    trow_v)
                    tlo = jnp.where(msk, tgt & c127, lane)
                    plsc.store_scatter(csrc, [thi, tlo], sv)
                    plsc.store_scatter(cdst, [thi, tlo], dv - lov)
                    plsc.store_scatter(cw, [thi, tlo], wv)
                    pcv = plsc.all_reduce_population_count(msk)
                    return cnt + pcv[0]
                cnt = lax.fori_loop(0, EB // LANES, step, cnt)

                nfire = cnt // GB
                fire_all(nfire, m_hbm)

                @pl.when(nfire > 0)
                def _move():
                    for h in range(GB // LANES):
                        sl = pl.ds(h * LANES, LANES)
                        csrc[0, sl] = csrc[nfire, sl]
                        cdst[0, sl] = cdst[nfire, sl]
                        cw[0, sl] = cw[nfire, sl]
                return cnt - nfire * GB

            cnt = lax.fori_loop(0, NB, bbody, jnp.int32(0))

            cntv = jnp.full((LANES,), cnt, jnp.int32)
            for t in range(GB // LANES):
                colv = jnp.full((LANES,), t * LANES, jnp.int32) + lane
                sel = colv >= cntv
                zhi = jnp.where(sel, zeros16, trow_v)
                zlo = jnp.where(sel, colv, lane)
                plsc.store_scatter(csrc, [zhi, zlo], zeros16)
                plsc.store_scatter(cdst, [zhi, zlo], zeros16)
                plsc.store_scatter(cw, [zhi, zlo], zf16)

            @pl.when(cnt > 0)
            def _tail():
                pltpu.async_copy(m_hbm.at[csrc.at[0]], rows0,
                                 g0sem).wait()
                scale(0, rows0)
                pltpu.sync_copy(rows0, acc.at[cdst.at[0]], add=True)

        for j in range(K_CHUNKS // NC):
            lo = (cid + NC * j) * C
            abase = sid * SPT

            pltpu.sync_copy(dense.at[pl.ds(lo + abase, SPT)],
                            acc.at[pl.ds(abase, SPT)])
            plsc.subcore_barrier()

            for (m_hbm, ep_hbm) in rels:
                scan_rel(m_hbm, ep_hbm, lo)
            plsc.subcore_barrier()

            pltpu.sync_copy(acc.at[pl.ds(abase, SPT)],
                            out.at[pl.ds(lo + abase, SPT)])
            plsc.subcore_barrier()

    return pl.kernel(
        body,
        out_type=jax.ShapeDtypeStruct((NP, H), jnp.float32),
        mesh=mesh,
        scratch_types=scratch,
        compiler_params=pltpu.CompilerParams(needs_layout_passes=False),
    )


_mm4 = _make_mm(4)
_mm2 = _make_mm(2)
_sc3 = _make_sc(3)
_sc1 = _make_sc(1)


def _pad_x(x):
    return jnp.pad(x, ((0, NP - N), (0, 0)))


def _pack_edges(ei, ew):
    pad = EP - E
    s = jnp.concatenate([ei[0], jnp.zeros((pad,), jnp.int32)])
    d = jnp.concatenate([ei[1], jnp.full((pad,), jnp.int32(1 << 30))])
    w = lax.bitcast_convert_type(
        jnp.concatenate([ew, jnp.zeros((pad,), jnp.float32)]), jnp.int32)
    blk = jnp.stack(
        [s.reshape(-1, EB), d.reshape(-1, EB), w.reshape(-1, EB)], axis=1)
    return blk.reshape(-1)


def kernel(x_app, x_sys, x_bnd, x_cmp, ei_app_sys, ew_app_sys, ei_sys_app,
           ew_sys_app, ei_app_bnd, ew_app_bnd, ei_bnd_app, ew_bnd_app,
           ei_app_cmp, ew_app_cmp, ei_cmp_app, ew_cmp_app, W_self_app,
           b_self_app, W_self_sys, b_self_sys, W_self_bnd, b_self_bnd,
           W_self_cmp, b_self_cmp, Wn_app_sys, Ws_app_sys, b_app_sys,
           Wn_sys_app, Ws_sys_app, b_sys_app, Wn_app_bnd, Ws_app_bnd,
           b_app_bnd, Wn_bnd_app, Ws_bnd_app, b_bnd_app, Wn_app_cmp,
           Ws_app_cmp, b_app_cmp, Wn_cmp_app, Ws_cmp_app, b_cmp_app):
    P_app = jnp.concatenate(
        [(W_self_app + Ws_sys_app + Ws_bnd_app + Ws_cmp_app).T,
         Wn_app_sys.T, Wn_app_bnd.T, Wn_app_cmp.T], axis=1)
    be_app = (b_self_app + b_sys_app + b_bnd_app + b_cmp_app).reshape(1, H)
    P_sys = jnp.concatenate([(W_self_sys + Ws_app_sys).T, Wn_sys_app.T],
                            axis=1)
    be_sys = (b_self_sys + b_app_sys).reshape(1, H)
    P_bnd = jnp.concatenate([(W_self_bnd + Ws_app_bnd).T, Wn_bnd_app.T],
                            axis=1)
    be_bnd = (b_self_bnd + b_app_bnd).reshape(1, H)
    P_cmp = jnp.concatenate([(W_self_cmp + Ws_app_cmp).T, Wn_cmp_app.T],
                            axis=1)
    be_cmp = (b_self_cmp + b_app_cmp).reshape(1, H)

    dense_app, m_app_sys, m_app_bnd, m_app_cmp = _mm4(
        _pad_x(x_app), P_app, be_app)
    dense_sys, m_sys_app = _mm2(_pad_x(x_sys), P_sys, be_sys)
    dense_bnd, m_bnd_app = _mm2(_pad_x(x_bnd), P_bnd, be_bnd)
    dense_cmp, m_cmp_app = _mm2(_pad_x(x_cmp), P_cmp, be_cmp)

    ep_sa = _pack_edges(ei_sys_app, ew_sys_app)
    ep_ba = _pack_edges(ei_bnd_app, ew_bnd_app)
    ep_ca = _pack_edges(ei_cmp_app, ew_cmp_app)
    ep_as = _pack_edges(ei_app_sys, ew_app_sys)
    ep_ab = _pack_edges(ei_app_bnd, ew_app_bnd)
    ep_ac = _pack_edges(ei_app_cmp, ew_app_cmp)

    out_app = _sc3(dense_app, m_sys_app, ep_sa, m_bnd_app, ep_ba,
                   m_cmp_app, ep_ca)
    out_sys = _sc1(dense_sys, m_app_sys, ep_as)
    out_bnd = _sc1(dense_bnd, m_app_bnd, ep_ab)
    out_cmp = _sc1(dense_cmp, m_app_cmp, ep_ac)

    return (out_app[:N], out_sys[:N], out_bnd[:N], out_cmp[:N])


# speedup vs baseline: 1.0299x; 1.0018x over previous
import functools

import jax
import jax.numpy as jnp
from jax import lax
from jax.experimental import pallas as pl
from jax.experimental.pallas import tpu as pltpu
from jax.experimental.pallas import tpu_sc as plsc

H = 128
N = 50000
E = 500000

NC = 2
NS = 16
LANES = 16

K_CHUNKS = 4
C = 12544
NP = K_CHUNKS * C

EP = 524288
S_TILE = EP // NS
EB = 1024
NB = S_TILE // EB
GB = 64
SH = GB.bit_length() - 1
NROWS = (EB + GB) // GB
TROW = NROWS
WB = 56
SPT = C // NS


def _mm_body(k, x_ref, p_ref, b_ref, *outs):
    acc = jnp.dot(x_ref[...], p_ref[...], preferred_element_type=jnp.float32)
    outs[0][...] = acc[:, :H] + b_ref[...]
    for j in range(1, k):
        outs[j][...] = acc[:, j * H:(j + 1) * H]


def _make_mm(k):
    mblk = 1024
    return pl.pallas_call(
        functools.partial(_mm_body, k),
        grid=(NP // mblk,),
        in_specs=[
            pl.BlockSpec((mblk, H), lambda i: (i, 0)),
            pl.BlockSpec((H, k * H), lambda i: (0, 0)),
            pl.BlockSpec((1, H), lambda i: (0, 0)),
        ],
        out_specs=[pl.BlockSpec((mblk, H), lambda i: (i, 0))] * k,
        out_shape=[jax.ShapeDtypeStruct((NP, H), jnp.float32)] * k,
    )


def _make_sc(n_rels):
    mesh = plsc.VectorSubcoreMesh(
        core_axis_name="c", subcore_axis_name="s",
        num_cores=NC, num_subcores=NS)
    scratch = [
        pltpu.VMEM_SHARED((C, H), jnp.float32),
        pltpu.VMEM((3 * EB,), jnp.int32),
        pltpu.VMEM((3 * EB,), jnp.int32),
        pltpu.VMEM((NROWS + 1, GB), jnp.int32),
        pltpu.VMEM((NROWS + 1, GB), jnp.int32),
        pltpu.VMEM((NROWS + 1, GB + LANES), jnp.float32),
        pltpu.VMEM((GB, H), jnp.float32),
        pltpu.VMEM((GB, H), jnp.float32),
        pltpu.SemaphoreType.DMA,
        pltpu.SemaphoreType.DMA,
        pltpu.SemaphoreType.DMA,
        pltpu.SemaphoreType.DMA,
        pltpu.SemaphoreType.DMA,
        pltpu.SemaphoreType.DMA,
    ]

    def body(*refs):
        dense = refs[0]
        rels = [refs[1 + 2 * j:3 + 2 * j] for j in range(n_rels)]
        out = refs[1 + 2 * n_rels]
        (acc, ebuf0, ebuf1, csrc, cdst, cw, rows0, rows1,
         g0sem, g1sem, s0sem, s1sem, e0sem, e1sem) = refs[2 + 2 * n_rels:]

        cid = lax.axis_index("c")
        sid = lax.axis_index("s")
        zf16 = jnp.zeros((LANES,), jnp.float32)

        def scale(b, rbuf):
            def srow(g2, _):
                for u in range(2):
                    g = g2 * 2 + u
                    wv16 = cw[b, pl.ds(g, LANES)]
                    wsp = jnp.full((LANES,), wv16[0], jnp.float32)
                    for h in range(H // LANES):
                        sl = pl.ds(h * LANES, LANES)
                        rbuf[g, sl] = rbuf[g, sl] * wsp
                return 0
            lax.fori_loop(0, GB // 2, srow, 0)

        def fire_all(nfire, m_hbm):
            @pl.when(nfire > 0)
            def _():
                pltpu.async_copy(m_hbm.at[csrc.at[0]], rows0, g0sem)

            def pair(k, _):
                b0 = 2 * k
                pltpu.make_async_copy(m_hbm.at[csrc.at[b0]], rows0,
                                      g0sem).wait()

                @pl.when(b0 + 1 < nfire)
                def _():
                    @pl.when(b0 >= 1)
                    def _():
                        pltpu.make_async_copy(
                            rows1, acc.at[cdst.at[0]], s1sem).wait()
                    pltpu.async_copy(m_hbm.at[csrc.at[b0 + 1]], rows1,
                                     g1sem)
                scale(b0, rows0)
                pltpu.async_copy(rows0, acc.at[cdst.at[b0]], s0sem,
                                 add=True)

                @pl.when(b0 + 1 < nfire)
                def _():
                    pltpu.make_async_copy(m_hbm.at[csrc.at[b0 + 1]],
                                          rows1, g1sem).wait()

                    @pl.when(b0 + 2 < nfire)
                    def _():
                        pltpu.make_async_copy(
                            rows0, acc.at[cdst.at[0]], s0sem).wait()
                        pltpu.async_copy(m_hbm.at[csrc.at[b0 + 2]],
                                         rows0, g0sem)
                    scale(b0 + 1, rows1)
                    pltpu.async_copy(rows1, acc.at[cdst.at[b0 + 1]],
                                     s1sem, add=True)
                return 0
            lax.fori_loop(0, (nfire + 1) // 2, pair, 0)

            @pl.when(nfire >= 1)
            def _():
                pltpu.make_async_copy(rows0, acc.at[cdst.at[0]],
                                      s0sem).wait()

            @pl.when(nfire >= 2)
            def _():
                pltpu.make_async_copy(rows1, acc.at[cdst.at[0]],
                                      s1sem).wait()

        lane = lax.iota(jnp.int32, LANES)
        ones16 = jnp.ones((LANES,), jnp.int32)
        zeros16 = jnp.zeros((LANES,), jnp.int32)
        trow_v = jnp.full((LANES,), TROW, jnp.int32)
        c127 = jnp.full((LANES,), GB - 1, jnp.int32)

        def scan_rel(m_hbm, ep_hbm, lo):
            lov = jnp.full((LANES,), lo, jnp.int32)
            hiv = lov + jnp.full((LANES,), C, jnp.int32)

            def scan_batch(b, ebuf, cnt):
                base = (sid * NB + b) * (3 * EB)
                pltpu.sync_copy(ep_hbm.at[pl.ds(base, 3 * EB)], ebuf)
                def step(i, cnt):
                    off = i * LANES
                    sv = ebuf[pl.ds(off, LANES)]
                    dv = ebuf[pl.ds(EB + off, LANES)]
                    wv = plsc.bitcast(ebuf[pl.ds(2 * EB + off, LANES)],
                                      jnp.float32)
                    msk = (dv >= lov) & (dv < hiv)
                    pc = plsc.cumsum(msk.astype(jnp.int32))
                    cntv = jnp.full((LANES,), cnt, jnp.int32)
                    tgt = jnp.where(msk, cntv + pc - ones16, zeros16)
                    thi = jnp.where(msk, lax.shift_right_logical(tgt, SH),
                                    trow_v)
                    tlo = jnp.where(msk, tgt & c127, lane)
                    plsc.store_scatter(csrc, [thi, tlo], sv)
                    plsc.store_scatter(cdst, [thi, tlo], dv - lov)
                    plsc.store_scatter(cw, [thi, tlo], wv)
                    pcv = plsc.all_reduce_population_count(msk)
                    return cnt + pcv[0]
                cnt = lax.fori_loop(0, EB // LANES, step, cnt)

                nfire = cnt // GB
                fire_all(nfire, m_hbm)

                @pl.when(nfire > 0)
                def _move():
                    for h in range(GB // LANES):
                        sl = pl.ds(h * LANES, LANES)
                        csrc[0, sl] = csrc[nfire, sl]
                        cdst[0, sl] = cdst[nfire, sl]
                        cw[0, sl] = cw[nfire, sl]
                return cnt - nfire * GB

            def bbody(b, cnt):
                return scan_batch(b, ebuf0, cnt)

            cnt = lax.fori_loop(0, NB, bbody, jnp.int32(0))

            cntv = jnp.full((LANES,), cnt, jnp.int32)
            for t in range(GB // LANES):
                colv = jnp.full((LANES,), t * LANES, jnp.int32) + lane
                sel = colv >= cntv
                zhi = jnp.where(sel, zeros16, trow_v)
                zlo = jnp.where(sel, colv, lane)
                plsc.store_scatter(csrc, [zhi, zlo], zeros16)
                plsc.store_scatter(cdst, [zhi, zlo], zeros16)
                plsc.store_scatter(cw, [zhi, zlo], zf16)

            @pl.when(cnt > 0)
            def _tail():
                pltpu.async_copy(m_hbm.at[csrc.at[0]], rows0,
                                 g0sem).wait()
                scale(0, rows0)
                pltpu.sync_copy(rows0, acc.at[cdst.at[0]], add=True)

        for j in range(K_CHUNKS // NC):
            lo = (cid + NC * j) * C
            abase = sid * SPT

            pltpu.sync_copy(dense.at[pl.ds(lo + abase, SPT)],
                            acc.at[pl.ds(abase, SPT)])
            plsc.subcore_barrier()

            for (m_hbm, ep_hbm) in rels:
                scan_rel(m_hbm, ep_hbm, lo)
            plsc.subcore_barrier()

            pltpu.sync_copy(acc.at[pl.ds(abase, SPT)],
                            out.at[pl.ds(lo + abase, SPT)])
            plsc.subcore_barrier()

    return pl.kernel(
        body,
        out_type=jax.ShapeDtypeStruct((NP, H), jnp.float32),
        mesh=mesh,
        scratch_types=scratch,
        compiler_params=pltpu.CompilerParams(needs_layout_passes=False),
    )


_mm4 = _make_mm(4)
_mm2 = _make_mm(2)
_sc3 = _make_sc(3)
_sc1 = _make_sc(1)


def _pad_x(x):
    return jnp.pad(x, ((0, NP - N), (0, 0)))


def _pack_edges(ei, ew):
    pad = EP - E
    s = jnp.concatenate([ei[0], jnp.zeros((pad,), jnp.int32)])
    d = jnp.concatenate([ei[1], jnp.full((pad,), jnp.int32(1 << 30))])
    w = lax.bitcast_convert_type(
        jnp.concatenate([ew, jnp.zeros((pad,), jnp.float32)]), jnp.int32)
    blk = jnp.stack(
        [s.reshape(-1, EB), d.reshape(-1, EB), w.reshape(-1, EB)], axis=1)
    return blk.reshape(-1)


def kernel(x_app, x_sys, x_bnd, x_cmp, ei_app_sys, ew_app_sys, ei_sys_app,
           ew_sys_app, ei_app_bnd, ew_app_bnd, ei_bnd_app, ew_bnd_app,
           ei_app_cmp, ew_app_cmp, ei_cmp_app, ew_cmp_app, W_self_app,
           b_self_app, W_self_sys, b_self_sys, W_self_bnd, b_self_bnd,
           W_self_cmp, b_self_cmp, Wn_app_sys, Ws_app_sys, b_app_sys,
           Wn_sys_app, Ws_sys_app, b_sys_app, Wn_app_bnd, Ws_app_bnd,
           b_app_bnd, Wn_bnd_app, Ws_bnd_app, b_bnd_app, Wn_app_cmp,
           Ws_app_cmp, b_app_cmp, Wn_cmp_app, Ws_cmp_app, b_cmp_app):
    P_app = jnp.concatenate(
        [(W_self_app + Ws_sys_app + Ws_bnd_app + Ws_cmp_app).T,
         Wn_app_sys.T, Wn_app_bnd.T, Wn_app_cmp.T], axis=1)
    be_app = (b_self_app + b_sys_app + b_bnd_app + b_cmp_app).reshape(1, H)
    P_sys = jnp.concatenate([(W_self_sys + Ws_app_sys).T, Wn_sys_app.T],
                            axis=1)
    be_sys = (b_self_sys + b_app_sys).reshape(1, H)
    P_bnd = jnp.concatenate([(W_self_bnd + Ws_app_bnd).T, Wn_bnd_app.T],
                            axis=1)
    be_bnd = (b_self_bnd + b_app_bnd).reshape(1, H)
    P_cmp = jnp.concatenate([(W_self_cmp + Ws_app_cmp).T, Wn_cmp_app.T],
                            axis=1)
    be_cmp = (b_self_cmp + b_app_cmp).reshape(1, H)

    dense_app, m_app_sys, m_app_bnd, m_app_cmp = _mm4(
        _pad_x(x_app), P_app, be_app)
    dense_sys, m_sys_app = _mm2(_pad_x(x_sys), P_sys, be_sys)
    dense_bnd, m_bnd_app = _mm2(_pad_x(x_bnd), P_bnd, be_bnd)
    dense_cmp, m_cmp_app = _mm2(_pad_x(x_cmp), P_cmp, be_cmp)

    ep_sa = _pack_edges(ei_sys_app, ew_sys_app)
    ep_ba = _pack_edges(ei_bnd_app, ew_bnd_app)
    ep_ca = _pack_edges(ei_cmp_app, ew_cmp_app)
    ep_as = _pack_edges(ei_app_sys, ew_app_sys)
    ep_ab = _pack_edges(ei_app_bnd, ew_app_bnd)
    ep_ac = _pack_edges(ei_app_cmp, ew_app_cmp)

    out_app = _sc3(dense_app, m_sys_app, ep_sa, m_bnd_app, ep_ba,
                   m_cmp_app, ep_ca)
    out_sys = _sc1(dense_sys, m_app_sys, ep_as)
    out_bnd = _sc1(dense_bnd, m_app_bnd, ep_ab)
    out_cmp = _sc1(dense_cmp, m_app_cmp, ep_ac)

    return (out_app[:N], out_sys[:N], out_bnd[:N], out_cmp[:N])
